# Initial kernel scaffold; baseline (speedup 1.0000x reference)
import jax
import jax.numpy as jnp
from jax.experimental import pallas as pl


def kernel(X, Y, edge_index, W):
    raise NotImplementedError("write your pallas kernel here")


# speedup vs baseline: 3.1923x; 3.1923x over previous
import functools

import jax
import jax.numpy as jnp
from jax import lax
from jax.experimental import pallas as pl
from jax.experimental.pallas import tpu as pltpu
from jax.experimental.pallas import tpu_sc as plsc

D = 128
BLOCK_E = 2000
CHUNK = 512
SUB = CHUNK // 128


def _matmul_body(y_ref, w_ref, out_ref):
    out_ref[...] = lax.dot_general(
        y_ref[...], w_ref[...],
        dimension_numbers=(((1,), (1,)), ((), ())),
        preferred_element_type=jnp.float32)


def _tc_matmul(Y, W):
    E = Y.shape[0]
    return pl.pallas_call(
        _matmul_body,
        grid=(E // BLOCK_E,),
        in_specs=[
            pl.BlockSpec((BLOCK_E, D), lambda i: (i, 0)),
            pl.BlockSpec((D, D), lambda i: (0, 0)),
        ],
        out_specs=pl.BlockSpec((BLOCK_E, D), lambda i: (i, 0)),
        out_shape=jax.ShapeDtypeStruct((E, D), jnp.float32),
    )(Y, W)


@functools.partial(jax.jit, static_argnums=())
def _sc_scatter(X, Yp, src2d):
    n_nodes = X.shape[0]
    E = Yp.shape[0]
    info = plsc.get_sparse_core_info()
    nc, ns = info.num_cores, info.num_subcores
    half = D // nc
    rows_per_tile = n_nodes // ns
    n_chunks = E // CHUNK
    q, r = divmod(n_chunks, ns)

    mesh = plsc.VectorSubcoreMesh(core_axis_name="c", subcore_axis_name="s")

    @functools.partial(
        pl.kernel,
        mesh=mesh,
        compiler_params=pltpu.CompilerParams(use_tc_tiling_on_sc=False),
        out_type=jax.ShapeDtypeStruct((n_nodes, D), jnp.float32),
        scratch_types=[
            pltpu.VMEM_SHARED((n_nodes, half), jnp.float32),
            pltpu.VMEM((SUB, 128), jnp.int32),
            pltpu.VMEM((CHUNK, half), jnp.float32),
        ],
    )
    def scatter_kernel(x_hbm, yp_hbm, src_hbm, out_hbm, acc_sh, idx_v, rows_v):
        c = lax.axis_index("c")
        s = lax.axis_index("s")
        r0 = s * rows_per_tile
        col0 = c * half

        pltpu.sync_copy(
            x_hbm.at[pl.ds(r0, rows_per_tile), pl.ds(col0, half)],
            acc_sh.at[pl.ds(r0, rows_per_tile)])
        plsc.subcore_barrier()

        nk = jnp.where(s < r, q + 1, q)

        def body(k, carry):
            ch = s + k * ns
            base = ch * CHUNK
            pltpu.sync_copy(src_hbm.at[pl.ds(ch * SUB, SUB)], idx_v)
            pltpu.sync_copy(
                yp_hbm.at[pl.ds(base, CHUNK), pl.ds(col0, half)], rows_v)
            for j in range(SUB):
                pltpu.sync_copy(
                    rows_v.at[pl.ds(j * 128, 128)],
                    acc_sh.at[idx_v.at[j]],
                    add=True)
            return carry

        lax.fori_loop(0, nk, body, jnp.int32(0))
        plsc.subcore_barrier()

        pltpu.sync_copy(
            acc_sh.at[pl.ds(r0, rows_per_tile)],
            out_hbm.at[pl.ds(r0, rows_per_tile), pl.ds(col0, half)])

    return scatter_kernel(X, Yp, src2d)


def kernel(X, Y, edge_index, W):
    Yp = _tc_matmul(Y, W)
    src2d = edge_index[:, 0].reshape(-1, 128)
    Xp = _sc_scatter(X, Yp, src2d)
    return (Xp, Yp)


# speedup vs baseline: 3.9345x; 1.2325x over previous
import functools

import jax
import jax.numpy as jnp
from jax import lax
from jax.experimental import pallas as pl
from jax.experimental.pallas import tpu as pltpu
from jax.experimental.pallas import tpu_sc as plsc

D = 128
BLOCK_E = 2000
CHUNK = 512
SUB = CHUNK // 128


def _matmul_body(y_ref, w_ref, out_ref):
    out_ref[...] = lax.dot_general(
        y_ref[...], w_ref[...],
        dimension_numbers=(((1,), (1,)), ((), ())),
        preferred_element_type=jnp.float32)


def _tc_matmul(Y, W):
    E = Y.shape[0]
    return pl.pallas_call(
        _matmul_body,
        grid=(E // BLOCK_E,),
        in_specs=[
            pl.BlockSpec((BLOCK_E, D), lambda i: (i, 0)),
            pl.BlockSpec((D, D), lambda i: (0, 0)),
        ],
        out_specs=pl.BlockSpec((BLOCK_E, D), lambda i: (i, 0)),
        out_shape=jax.ShapeDtypeStruct((E, D), jnp.float32),
    )(Y, W)


@functools.partial(jax.jit, static_argnums=())
def _sc_scatter(X, Yp, src2d):
    n_nodes = X.shape[0]
    E = Yp.shape[0]
    info = plsc.get_sparse_core_info()
    nc, ns = info.num_cores, info.num_subcores
    half = D // nc
    rows_per_tile = n_nodes // ns
    n_chunks = E // CHUNK
    q, r = divmod(n_chunks, ns)

    mesh = plsc.VectorSubcoreMesh(core_axis_name="c", subcore_axis_name="s")

    @functools.partial(
        pl.kernel,
        mesh=mesh,
        compiler_params=pltpu.CompilerParams(use_tc_tiling_on_sc=False),
        out_type=jax.ShapeDtypeStruct((n_nodes, D), jnp.float32),
        scratch_types=[
            pltpu.VMEM_SHARED((n_nodes, half), jnp.float32),
            pltpu.VMEM((2, SUB, 128), jnp.int32),
            pltpu.VMEM((2, CHUNK, half), jnp.float32),
            pltpu.SemaphoreType.DMA((2,)),
            pltpu.SemaphoreType.DMA((2,)),
        ],
    )
    def scatter_kernel(x_hbm, yp_hbm, src_hbm, out_hbm, acc_sh, idx_v, rows_v,
                       sem_i, sem_r):
        c = lax.axis_index("c")
        s = lax.axis_index("s")
        r0 = s * rows_per_tile
        col0 = c * half

        pltpu.sync_copy(
            x_hbm.at[pl.ds(r0, rows_per_tile), pl.ds(col0, half)],
            acc_sh.at[pl.ds(r0, rows_per_tile)])
        plsc.subcore_barrier()

        nk = jnp.where(s < r, q + 1, q)

        def fetch(k, b):
            ch = s + k * ns
            pltpu.async_copy(
                src_hbm.at[pl.ds(ch * SUB, SUB)], idx_v.at[b], sem_i.at[b])
            pltpu.async_copy(
                yp_hbm.at[pl.ds(ch * CHUNK, CHUNK), pl.ds(col0, half)],
                rows_v.at[b], sem_r.at[b])

        fetch(jnp.int32(0), jnp.int32(0))

        def body(k, carry):
            b = lax.rem(k, 2)

            @pl.when(k + 1 < nk)
            def _():
                fetch(k + 1, lax.rem(k + 1, 2))

            ch = s + k * ns
            pltpu.make_async_copy(
                src_hbm.at[pl.ds(ch * SUB, SUB)], idx_v.at[b],
                sem_i.at[b]).wait()
            pltpu.make_async_copy(
                yp_hbm.at[pl.ds(ch * CHUNK, CHUNK), pl.ds(col0, half)],
                rows_v.at[b], sem_r.at[b]).wait()
            for j in range(SUB):
                pltpu.sync_copy(
                    rows_v.at[b, pl.ds(j * 128, 128)],
                    acc_sh.at[idx_v.at[b, j]],
                    add=True)
            return carry

        lax.fori_loop(0, nk, body, jnp.int32(0))
        plsc.subcore_barrier()

        pltpu.sync_copy(
            acc_sh.at[pl.ds(r0, rows_per_tile)],
            out_hbm.at[pl.ds(r0, rows_per_tile), pl.ds(col0, half)])

    return scatter_kernel(X, Yp, src2d)


def kernel(X, Y, edge_index, W):
    Yp = _tc_matmul(Y, W)
    src2d = edge_index[:, 0].reshape(-1, 128)
    Xp = _sc_scatter(X, Yp, src2d)
    return (Xp, Yp)


# speedup vs baseline: 5.0716x; 1.2890x over previous
import functools

import jax
import jax.numpy as jnp
from jax import lax
from jax.experimental import pallas as pl
from jax.experimental.pallas import tpu as pltpu
from jax.experimental.pallas import tpu_sc as plsc

D = 128
BLOCK_E = 8000
CHUNK = 512
SUB = CHUNK // 128


def _matmul_body(y_ref, w_ref, out_ref):
    out_ref[...] = lax.dot_general(
        y_ref[...], w_ref[...],
        dimension_numbers=(((1,), (1,)), ((), ())),
        preferred_element_type=jnp.float32)


def _tc_matmul(Y, W):
    E = Y.shape[0]
    return pl.pallas_call(
        _matmul_body,
        grid=(E // BLOCK_E,),
        in_specs=[
            pl.BlockSpec((BLOCK_E, D), lambda i: (i, 0)),
            pl.BlockSpec((D, D), lambda i: (0, 0)),
        ],
        out_specs=pl.BlockSpec((BLOCK_E, D), lambda i: (i, 0)),
        out_shape=jax.ShapeDtypeStruct((E, D), jnp.float32),
    )(Y, W)


@functools.partial(jax.jit, static_argnums=())
def _sc_scatter(X, Yp, src2d):
    n_nodes = X.shape[0]
    E = Yp.shape[0]
    info = plsc.get_sparse_core_info()
    nc, ns = info.num_cores, info.num_subcores
    half = D // nc
    rows_per_tile = n_nodes // ns
    n_chunks = E // CHUNK
    q, r = divmod(n_chunks, ns)

    mesh = plsc.VectorSubcoreMesh(core_axis_name="c", subcore_axis_name="s")

    @functools.partial(
        pl.kernel,
        mesh=mesh,
        compiler_params=pltpu.CompilerParams(use_tc_tiling_on_sc=False),
        out_type=jax.ShapeDtypeStruct((n_nodes, D), jnp.float32),
        scratch_types=[
            pltpu.VMEM_SHARED((n_nodes, half), jnp.float32),
            pltpu.VMEM((2, SUB, 128), jnp.int32),
            pltpu.VMEM((2, CHUNK, half), jnp.float32),
            pltpu.SemaphoreType.DMA((2,)),
            pltpu.SemaphoreType.DMA((2,)),
        ],
    )
    def scatter_kernel(x_hbm, yp_hbm, src_hbm, out_hbm, acc_sh, idx_v, rows_v,
                       sem_i, sem_r):
        c = lax.axis_index("c")
        s = lax.axis_index("s")
        r0 = s * rows_per_tile
        col0 = c * half

        pltpu.sync_copy(
            x_hbm.at[pl.ds(r0, rows_per_tile), pl.ds(col0, half)],
            acc_sh.at[pl.ds(r0, rows_per_tile)])
        plsc.subcore_barrier()

        nk = jnp.where(s < r, q + 1, q)

        def fetch(k, b):
            ch = s + k * ns
            pltpu.async_copy(
                src_hbm.at[pl.ds(ch * SUB, SUB)], idx_v.at[b], sem_i.at[b])
            pltpu.async_copy(
                yp_hbm.at[pl.ds(ch * CHUNK, CHUNK), pl.ds(col0, half)],
                rows_v.at[b], sem_r.at[b])

        fetch(jnp.int32(0), jnp.int32(0))

        def body(k, carry):
            b = lax.rem(k, 2)

            @pl.when(k + 1 < nk)
            def _():
                fetch(k + 1, lax.rem(k + 1, 2))

            ch = s + k * ns
            pltpu.make_async_copy(
                src_hbm.at[pl.ds(ch * SUB, SUB)], idx_v.at[b],
                sem_i.at[b]).wait()
            pltpu.make_async_copy(
                yp_hbm.at[pl.ds(ch * CHUNK, CHUNK), pl.ds(col0, half)],
                rows_v.at[b], sem_r.at[b]).wait()
            for j in range(SUB):
                pltpu.sync_copy(
                    rows_v.at[b, pl.ds(j * 128, 128)],
                    acc_sh.at[idx_v.at[b, j]],
                    add=True)
            return carry

        lax.fori_loop(0, nk, body, jnp.int32(0))
        plsc.subcore_barrier()

        pltpu.sync_copy(
            acc_sh.at[pl.ds(r0, rows_per_tile)],
            out_hbm.at[pl.ds(r0, rows_per_tile), pl.ds(col0, half)])

    return scatter_kernel(X, Yp, src2d)


def kernel(X, Y, edge_index, W):
    Yp = _tc_matmul(Y, W)
    src2d = edge_index[:, 0].reshape(-1, 128)
    Xp = _sc_scatter(X, Yp, src2d)
    return (Xp, Yp)


# speedup vs baseline: 5.1320x; 1.0119x over previous
import functools

import jax
import jax.numpy as jnp
from jax import lax
from jax.experimental import pallas as pl
from jax.experimental.pallas import tpu as pltpu
from jax.experimental.pallas import tpu_sc as plsc

D = 128
BLOCK_E = 16000
CHUNK = 512
SUB = CHUNK // 128


def _matmul_body(y_ref, w_ref, out_ref):
    out_ref[...] = lax.dot_general(
        y_ref[...], w_ref[...],
        dimension_numbers=(((1,), (1,)), ((), ())),
        preferred_element_type=jnp.float32)


def _tc_matmul(Y, W):
    E = Y.shape[0]
    return pl.pallas_call(
        _matmul_body,
        grid=(E // BLOCK_E,),
        in_specs=[
            pl.BlockSpec((BLOCK_E, D), lambda i: (i, 0)),
            pl.BlockSpec((D, D), lambda i: (0, 0)),
        ],
        out_specs=pl.BlockSpec((BLOCK_E, D), lambda i: (i, 0)),
        out_shape=jax.ShapeDtypeStruct((E, D), jnp.float32),
    )(Y, W)


@functools.partial(jax.jit, static_argnums=())
def _sc_scatter(X, Yp, src2d):
    n_nodes = X.shape[0]
    E = Yp.shape[0]
    info = plsc.get_sparse_core_info()
    nc, ns = info.num_cores, info.num_subcores
    half = D // nc
    rows_per_tile = n_nodes // ns
    n_chunks = E // CHUNK
    q, r = divmod(n_chunks, ns)

    mesh = plsc.VectorSubcoreMesh(core_axis_name="c", subcore_axis_name="s")

    @functools.partial(
        pl.kernel,
        mesh=mesh,
        compiler_params=pltpu.CompilerParams(use_tc_tiling_on_sc=False),
        out_type=jax.ShapeDtypeStruct((n_nodes, D), jnp.float32),
        scratch_types=[
            pltpu.VMEM_SHARED((n_nodes, half), jnp.float32),
            pltpu.VMEM((2, SUB, 128), jnp.int32),
            pltpu.VMEM((2, CHUNK, half), jnp.float32),
            pltpu.SemaphoreType.DMA((2,)),
            pltpu.SemaphoreType.DMA((2,)),
        ],
    )
    def scatter_kernel(x_hbm, yp_hbm, src_hbm, out_hbm, acc_sh, idx_v, rows_v,
                       sem_i, sem_r):
        c = lax.axis_index("c")
        s = lax.axis_index("s")
        r0 = s * rows_per_tile
        col0 = c * half

        pltpu.sync_copy(
            x_hbm.at[pl.ds(r0, rows_per_tile), pl.ds(col0, half)],
            acc_sh.at[pl.ds(r0, rows_per_tile)])
        plsc.subcore_barrier()

        nk = jnp.where(s < r, q + 1, q)

        def fetch(k, b):
            ch = s + k * ns
            pltpu.async_copy(
                src_hbm.at[pl.ds(ch * SUB, SUB)], idx_v.at[b], sem_i.at[b])
            pltpu.async_copy(
                yp_hbm.at[pl.ds(ch * CHUNK, CHUNK), pl.ds(col0, half)],
                rows_v.at[b], sem_r.at[b])

        fetch(jnp.int32(0), jnp.int32(0))

        def body(k, carry):
            b = lax.rem(k, 2)

            @pl.when(k + 1 < nk)
            def _():
                fetch(k + 1, lax.rem(k + 1, 2))

            ch = s + k * ns
            pltpu.make_async_copy(
                src_hbm.at[pl.ds(ch * SUB, SUB)], idx_v.at[b],
                sem_i.at[b]).wait()
            pltpu.make_async_copy(
                yp_hbm.at[pl.ds(ch * CHUNK, CHUNK), pl.ds(col0, half)],
                rows_v.at[b], sem_r.at[b]).wait()
            for j in range(SUB):
                pltpu.sync_copy(
                    rows_v.at[b, pl.ds(j * 128, 128)],
                    acc_sh.at[idx_v.at[b, j]],
                    add=True)
            return carry

        lax.fori_loop(0, nk, body, jnp.int32(0))
        plsc.subcore_barrier()

        pltpu.sync_copy(
            acc_sh.at[pl.ds(r0, rows_per_tile)],
            out_hbm.at[pl.ds(r0, rows_per_tile), pl.ds(col0, half)])

    return scatter_kernel(X, Yp, src2d)


def kernel(X, Y, edge_index, W):
    Yp = _tc_matmul(Y, W)
    src2d = edge_index[:, 0].reshape(-1, 128)
    Xp = _sc_scatter(X, Yp, src2d)
    return (Xp, Yp)


# speedup vs baseline: 5.6772x; 1.1062x over previous
import functools

import jax
import jax.numpy as jnp
from jax import lax
from jax.experimental import pallas as pl
from jax.experimental.pallas import tpu as pltpu
from jax.experimental.pallas import tpu_sc as plsc

D = 128
BLOCK_E = 16000
CHUNK = 512
SUB = CHUNK // 128


def _matmul_body(y_ref, w_ref, out_ref):
    out_ref[...] = lax.dot_general(
        y_ref[...], w_ref[...],
        dimension_numbers=(((1,), (1,)), ((), ())),
        preferred_element_type=jnp.float32)


def _tc_matmul(Y, W):
    E = Y.shape[0]
    return pl.pallas_call(
        _matmul_body,
        grid=(E // BLOCK_E,),
        in_specs=[
            pl.BlockSpec((BLOCK_E, D), lambda i: (i, 0)),
            pl.BlockSpec((D, D), lambda i: (0, 0)),
        ],
        out_specs=pl.BlockSpec((BLOCK_E, D), lambda i: (i, 0)),
        out_shape=jax.ShapeDtypeStruct((E, D), jnp.float32),
    )(Y, W)


def _final_body(x_ref, agg_ref, w_ref, out_ref):
    out_ref[...] = x_ref[...] + lax.dot_general(
        agg_ref[...], w_ref[...],
        dimension_numbers=(((1,), (1,)), ((), ())),
        preferred_element_type=jnp.float32)


def _tc_final(X, agg, W, block_n=2000):
    n_nodes = X.shape[0]
    return pl.pallas_call(
        _final_body,
        grid=(n_nodes // block_n,),
        in_specs=[
            pl.BlockSpec((block_n, D), lambda i: (i, 0)),
            pl.BlockSpec((block_n, D), lambda i: (i, 0)),
            pl.BlockSpec((D, D), lambda i: (0, 0)),
        ],
        out_specs=pl.BlockSpec((block_n, D), lambda i: (i, 0)),
        out_shape=jax.ShapeDtypeStruct((n_nodes, D), jnp.float32),
    )(X, agg, W)


def _sc_scatter(zeros_half, Yraw, src2d, n_nodes):
    E = Yraw.shape[0]
    info = plsc.get_sparse_core_info()
    nc, ns = info.num_cores, info.num_subcores
    half = D // nc
    rows_per_tile = n_nodes // ns
    n_chunks = E // CHUNK
    q, r = divmod(n_chunks, ns)

    mesh = plsc.VectorSubcoreMesh(core_axis_name="c", subcore_axis_name="s")

    @functools.partial(
        pl.kernel,
        mesh=mesh,
        compiler_params=pltpu.CompilerParams(use_tc_tiling_on_sc=False),
        out_type=jax.ShapeDtypeStruct((n_nodes, D), jnp.float32),
        scratch_types=[
            pltpu.VMEM_SHARED((n_nodes, half), jnp.float32),
            pltpu.VMEM((2, SUB, 128), jnp.int32),
            pltpu.VMEM((2, CHUNK, half), jnp.float32),
            pltpu.SemaphoreType.DMA((2,)),
            pltpu.SemaphoreType.DMA((2,)),
        ],
    )
    def scatter_kernel(z_hbm, yp_hbm, src_hbm, out_hbm, acc_sh, idx_v, rows_v,
                       sem_i, sem_r):
        c = lax.axis_index("c")
        s = lax.axis_index("s")
        r0 = s * rows_per_tile
        col0 = c * half

        pltpu.sync_copy(
            z_hbm.at[pl.ds(r0, rows_per_tile)],
            acc_sh.at[pl.ds(r0, rows_per_tile)])
        plsc.subcore_barrier()

        nk = jnp.where(s < r, q + 1, q)

        def fetch(k, b):
            ch = s + k * ns
            pltpu.async_copy(
                src_hbm.at[pl.ds(ch * SUB, SUB)], idx_v.at[b], sem_i.at[b])
            pltpu.async_copy(
                yp_hbm.at[pl.ds(ch * CHUNK, CHUNK), pl.ds(col0, half)],
                rows_v.at[b], sem_r.at[b])

        fetch(jnp.int32(0), jnp.int32(0))

        def body(k, carry):
            b = lax.rem(k, 2)

            @pl.when(k + 1 < nk)
            def _():
                fetch(k + 1, lax.rem(k + 1, 2))

            ch = s + k * ns
            pltpu.make_async_copy(
                src_hbm.at[pl.ds(ch * SUB, SUB)], idx_v.at[b],
                sem_i.at[b]).wait()
            pltpu.make_async_copy(
                yp_hbm.at[pl.ds(ch * CHUNK, CHUNK), pl.ds(col0, half)],
                rows_v.at[b], sem_r.at[b]).wait()
            for j in range(SUB):
                pltpu.sync_copy(
                    rows_v.at[b, pl.ds(j * 128, 128)],
                    acc_sh.at[idx_v.at[b, j]],
                    add=True)
            return carry

        lax.fori_loop(0, nk, body, jnp.int32(0))
        plsc.subcore_barrier()

        pltpu.sync_copy(
            acc_sh.at[pl.ds(r0, rows_per_tile)],
            out_hbm.at[pl.ds(r0, rows_per_tile), pl.ds(col0, half)])

    return scatter_kernel(zeros_half, Yraw, src2d)


def kernel(X, Y, edge_index, W):
    n_nodes = X.shape[0]
    src2d = edge_index[:, 0].reshape(-1, 128)
    zeros_half = jnp.zeros((n_nodes, D // 2), jnp.float32)
    agg = _sc_scatter(zeros_half, Y, src2d, n_nodes)
    Yp = _tc_matmul(Y, W)
    Xp = _tc_final(X, agg, W)
    return (Xp, Yp)


# speedup vs baseline: 5.7459x; 1.0121x over previous
import functools

import jax
import jax.numpy as jnp
from jax import lax
from jax.experimental import pallas as pl
from jax.experimental.pallas import tpu as pltpu
from jax.experimental.pallas import tpu_sc as plsc

D = 128
BLOCK_E = 16000
CHUNK = 512
SUB = CHUNK // 128


def _matmul_body(y_ref, w_ref, out_ref):
    out_ref[...] = lax.dot_general(
        y_ref[...], w_ref[...],
        dimension_numbers=(((1,), (1,)), ((), ())),
        preferred_element_type=jnp.float32)


def _tc_matmul(Y, W):
    E = Y.shape[0]
    return pl.pallas_call(
        _matmul_body,
        grid=(E // BLOCK_E,),
        in_specs=[
            pl.BlockSpec((BLOCK_E, D), lambda i: (i, 0)),
            pl.BlockSpec((D, D), lambda i: (0, 0)),
        ],
        out_specs=pl.BlockSpec((BLOCK_E, D), lambda i: (i, 0)),
        out_shape=jax.ShapeDtypeStruct((E, D), jnp.float32),
    )(Y, W)


def _final_body(x_ref, agg_ref, w_ref, out_ref):
    out_ref[...] = x_ref[...] + lax.dot_general(
        agg_ref[...], w_ref[...],
        dimension_numbers=(((1,), (1,)), ((), ())),
        preferred_element_type=jnp.float32)


def _tc_final(X, agg, W, block_n=2000):
    n_nodes = X.shape[0]
    return pl.pallas_call(
        _final_body,
        grid=(n_nodes // block_n,),
        in_specs=[
            pl.BlockSpec((block_n, D), lambda i: (i, 0)),
            pl.BlockSpec((block_n, D), lambda i: (i, 0)),
            pl.BlockSpec((D, D), lambda i: (0, 0)),
        ],
        out_specs=pl.BlockSpec((block_n, D), lambda i: (i, 0)),
        out_shape=jax.ShapeDtypeStruct((n_nodes, D), jnp.float32),
    )(X, agg, W)


def _sc_scatter(Yraw, src2d, n_nodes):
    E = Yraw.shape[0]
    info = plsc.get_sparse_core_info()
    nc, ns = info.num_cores, info.num_subcores
    half = D // nc
    rows_per_tile = n_nodes // ns
    n_chunks = E // CHUNK
    q, r = divmod(n_chunks, ns)

    mesh = plsc.VectorSubcoreMesh(core_axis_name="c", subcore_axis_name="s")

    @functools.partial(
        pl.kernel,
        mesh=mesh,
        compiler_params=pltpu.CompilerParams(use_tc_tiling_on_sc=False),
        out_type=jax.ShapeDtypeStruct((n_nodes, D), jnp.float32),
        scratch_types=[
            pltpu.VMEM_SHARED((n_nodes, half), jnp.float32),
            pltpu.VMEM((2, SUB, 128), jnp.int32),
            pltpu.VMEM((2, CHUNK, half), jnp.float32),
            pltpu.SemaphoreType.DMA((2,)),
            pltpu.SemaphoreType.DMA((2,)),
        ],
    )
    def scatter_kernel(yp_hbm, src_hbm, out_hbm, acc_sh, idx_v, rows_v,
                       sem_i, sem_r):
        c = lax.axis_index("c")
        s = lax.axis_index("s")
        r0 = s * rows_per_tile
        col0 = c * half

        zv = jnp.zeros((16,), jnp.float32)
        lanes = half // 16

        def zb(i, carry):
            rows_v[0, lax.div(i, lanes), pl.ds(lax.rem(i, lanes) * 16, 16)] = zv
            return carry

        lax.fori_loop(0, 128 * lanes, zb, jnp.int32(0))
        for p in range(4):
            pltpu.sync_copy(rows_v.at[0, pl.ds(0, 128)],
                            acc_sh.at[pl.ds(r0 + p * 128, 128)])
        pltpu.sync_copy(
            rows_v.at[0, pl.ds(0, rows_per_tile - 512)],
            acc_sh.at[pl.ds(r0 + 512, rows_per_tile - 512)])
        plsc.subcore_barrier()

        nk = jnp.where(s < r, q + 1, q)

        def fetch(k, b):
            ch = s + k * ns
            pltpu.async_copy(
                src_hbm.at[pl.ds(ch * SUB, SUB)], idx_v.at[b], sem_i.at[b])
            pltpu.async_copy(
                yp_hbm.at[pl.ds(ch * CHUNK, CHUNK), pl.ds(col0, half)],
                rows_v.at[b], sem_r.at[b])

        fetch(jnp.int32(0), jnp.int32(0))

        def body(k, carry):
            b = lax.rem(k, 2)

            @pl.when(k + 1 < nk)
            def _():
                fetch(k + 1, lax.rem(k + 1, 2))

            ch = s + k * ns
            pltpu.make_async_copy(
                src_hbm.at[pl.ds(ch * SUB, SUB)], idx_v.at[b],
                sem_i.at[b]).wait()
            pltpu.make_async_copy(
                yp_hbm.at[pl.ds(ch * CHUNK, CHUNK), pl.ds(col0, half)],
                rows_v.at[b], sem_r.at[b]).wait()
            for j in range(SUB):
                pltpu.sync_copy(
                    rows_v.at[b, pl.ds(j * 128, 128)],
                    acc_sh.at[idx_v.at[b, j]],
                    add=True)
            return carry

        lax.fori_loop(0, nk, body, jnp.int32(0))
        plsc.subcore_barrier()

        pltpu.sync_copy(
            acc_sh.at[pl.ds(r0, rows_per_tile)],
            out_hbm.at[pl.ds(r0, rows_per_tile), pl.ds(col0, half)])

    return scatter_kernel(Yraw, src2d)


def kernel(X, Y, edge_index, W):
    n_nodes = X.shape[0]
    src2d = edge_index[:, 0].reshape(-1, 128)
    agg = _sc_scatter(Y, src2d, n_nodes)
    Yp = _tc_matmul(Y, W)
    Xp = _tc_final(X, agg, W)
    return (Xp, Yp)


# speedup vs baseline: 5.7619x; 1.0028x over previous
import functools

import jax
import jax.numpy as jnp
from jax import lax
from jax.experimental import pallas as pl
from jax.experimental.pallas import tpu as pltpu
from jax.experimental.pallas import tpu_sc as plsc

D = 128
BLOCK_E = 20000
CHUNK = 512
SUB = CHUNK // 128


def _matmul_body(y_ref, w_ref, out_ref):
    out_ref[...] = lax.dot_general(
        y_ref[...], w_ref[...],
        dimension_numbers=(((1,), (1,)), ((), ())),
        preferred_element_type=jnp.float32)


def _tc_matmul(Y, W):
    E = Y.shape[0]
    return pl.pallas_call(
        _matmul_body,
        grid=(E // BLOCK_E,),
        in_specs=[
            pl.BlockSpec((BLOCK_E, D), lambda i: (i, 0)),
            pl.BlockSpec((D, D), lambda i: (0, 0)),
        ],
        out_specs=pl.BlockSpec((BLOCK_E, D), lambda i: (i, 0)),
        out_shape=jax.ShapeDtypeStruct((E, D), jnp.float32),
    )(Y, W)


def _final_body(x_ref, agg_ref, w_ref, out_ref):
    out_ref[...] = x_ref[...] + lax.dot_general(
        agg_ref[...], w_ref[...],
        dimension_numbers=(((1,), (1,)), ((), ())),
        preferred_element_type=jnp.float32)


def _tc_final(X, agg, W, block_n=2000):
    n_nodes = X.shape[0]
    return pl.pallas_call(
        _final_body,
        grid=(n_nodes // block_n,),
        in_specs=[
            pl.BlockSpec((block_n, D), lambda i: (i, 0)),
            pl.BlockSpec((block_n, D), lambda i: (i, 0)),
            pl.BlockSpec((D, D), lambda i: (0, 0)),
        ],
        out_specs=pl.BlockSpec((block_n, D), lambda i: (i, 0)),
        out_shape=jax.ShapeDtypeStruct((n_nodes, D), jnp.float32),
    )(X, agg, W)


def _sc_scatter(Yraw, src2d, n_nodes):
    E = Yraw.shape[0]
    info = plsc.get_sparse_core_info()
    nc, ns = info.num_cores, info.num_subcores
    half = D // nc
    rows_per_tile = n_nodes // ns
    n_chunks = E // CHUNK
    q, r = divmod(n_chunks, ns)

    mesh = plsc.VectorSubcoreMesh(core_axis_name="c", subcore_axis_name="s")

    @functools.partial(
        pl.kernel,
        mesh=mesh,
        compiler_params=pltpu.CompilerParams(use_tc_tiling_on_sc=False),
        out_type=jax.ShapeDtypeStruct((n_nodes, D), jnp.float32),
        scratch_types=[
            pltpu.VMEM_SHARED((n_nodes, half), jnp.float32),
            pltpu.VMEM((2, SUB, 128), jnp.int32),
            pltpu.VMEM((2, CHUNK, half), jnp.float32),
            pltpu.SemaphoreType.DMA((2,)),
            pltpu.SemaphoreType.DMA((2,)),
        ],
    )
    def scatter_kernel(yp_hbm, src_hbm, out_hbm, acc_sh, idx_v, rows_v,
                       sem_i, sem_r):
        c = lax.axis_index("c")
        s = lax.axis_index("s")
        r0 = s * rows_per_tile
        col0 = c * half

        zv = jnp.zeros((16,), jnp.float32)
        lanes = half // 16

        def zb(i, carry):
            rows_v[0, lax.div(i, lanes), pl.ds(lax.rem(i, lanes) * 16, 16)] = zv
            return carry

        lax.fori_loop(0, 128 * lanes, zb, jnp.int32(0))
        for p in range(4):
            pltpu.sync_copy(rows_v.at[0, pl.ds(0, 128)],
                            acc_sh.at[pl.ds(r0 + p * 128, 128)])
        pltpu.sync_copy(
            rows_v.at[0, pl.ds(0, rows_per_tile - 512)],
            acc_sh.at[pl.ds(r0 + 512, rows_per_tile - 512)])
        plsc.subcore_barrier()

        nk = jnp.where(s < r, q + 1, q)

        def fetch(k, b):
            ch = s + k * ns
            pltpu.async_copy(
                src_hbm.at[pl.ds(ch * SUB, SUB)], idx_v.at[b], sem_i.at[b])
            pltpu.async_copy(
                yp_hbm.at[pl.ds(ch * CHUNK, CHUNK), pl.ds(col0, half)],
                rows_v.at[b], sem_r.at[b])

        fetch(jnp.int32(0), jnp.int32(0))

        def body(k, carry):
            b = lax.rem(k, 2)

            @pl.when(k + 1 < nk)
            def _():
                fetch(k + 1, lax.rem(k + 1, 2))

            ch = s + k * ns
            pltpu.make_async_copy(
                src_hbm.at[pl.ds(ch * SUB, SUB)], idx_v.at[b],
                sem_i.at[b]).wait()
            pltpu.make_async_copy(
                yp_hbm.at[pl.ds(ch * CHUNK, CHUNK), pl.ds(col0, half)],
                rows_v.at[b], sem_r.at[b]).wait()
            for j in range(SUB):
                pltpu.sync_copy(
                    rows_v.at[b, pl.ds(j * 128, 128)],
                    acc_sh.at[idx_v.at[b, j]],
                    add=True)
            return carry

        lax.fori_loop(0, nk, body, jnp.int32(0))
        plsc.subcore_barrier()

        pltpu.sync_copy(
            acc_sh.at[pl.ds(r0, rows_per_tile)],
            out_hbm.at[pl.ds(r0, rows_per_tile), pl.ds(col0, half)])

    return scatter_kernel(Yraw, src2d)


def kernel(X, Y, edge_index, W):
    n_nodes = X.shape[0]
    src2d = edge_index[:, 0].reshape(-1, 128)
    agg = _sc_scatter(Y, src2d, n_nodes)
    Yp = _tc_matmul(Y, W)
    Xp = _tc_final(X, agg, W)
    return (Xp, Yp)
